# baseline (speedup 1.0000x reference)
import functools

import jax
import jax.numpy as jnp
from jax import lax
from jax.experimental import pallas as pl
from jax.experimental.pallas import tpu as pltpu
from jax.experimental.pallas import tpu_sc as plsc

K_NEIGH = 16
M1 = 256
M3 = 256
TBL = 384
GATHER_CHUNK = 128
NUM_WORKERS = 32


def _mm(a, b):
    return lax.dot_general(
        a.astype(jnp.bfloat16), b.astype(jnp.bfloat16),
        (((1,), (0,)), ((), ())), preferred_element_type=jnp.float32)


def _pack(lo_bf16, hi_bf16):
    lo = lax.bitcast_convert_type(lo_bf16, jnp.int16).astype(jnp.int32)
    hi = lax.bitcast_convert_type(hi_bf16, jnp.int16).astype(jnp.int32)
    return (lo & jnp.int32(0xFFFF)) | (hi << 16)


def _unpack_lo(w):
    return lax.bitcast_convert_type(
        w.astype(jnp.int16), jnp.bfloat16).astype(jnp.float32)


def _unpack_hi(w):
    return lax.bitcast_convert_type(
        lax.shift_right_logical(w, 16).astype(jnp.int16),
        jnp.bfloat16).astype(jnp.float32)


def _stage1_body(xyzp_ref, xyzt_ref, feat_ref,
                 kW_ref, kb_ref, wq_ref, wk_ref, wv_ref, a1W_ref, a1b_ref,
                 tmain_ref, a_ref, idx_ref):
    n = xyzt_ref.shape[2]
    m = feat_ref.shape[1]
    b = pl.program_id(0)

    fb = feat_ref[0]
    x = _mm(fb, kW_ref[...]) + kb_ref[...]
    q = _mm(x, wq_ref[...])
    a_ref[0] = _mm(q, a1W_ref[...]) + a1b_ref[...]
    kk = _mm(x, wk_ref[...])
    bk = _mm(kk, a1W_ref[...])
    v = _mm(x, wv_ref[...])

    xb = xyzp_ref[0]
    xhi = xb.astype(jnp.bfloat16)
    xlo = (xb - xhi.astype(jnp.float32)).astype(jnp.bfloat16)
    tmain_ref[0] = jnp.concatenate(
        [_pack(bk.astype(jnp.bfloat16), v.astype(jnp.bfloat16)),
         _pack(xhi, xlo),
         jnp.zeros((m, TBL - 260), jnp.int32)], axis=1)

    xyz_i = xb[:, 0:3]
    xyz_jt = xyzt_ref[0][0:3, :]
    sq_i = jnp.sum(xyz_i * xyz_i, axis=1, keepdims=True)
    sq_j = jnp.sum(xyz_jt * xyz_jt, axis=0, keepdims=True)
    d2 = sq_i + sq_j - 2.0 * _mm(xyz_i, xyz_jt)

    lane = lax.broadcasted_iota(jnp.int32, (m, n), 1)
    sels = []
    for _ in range(K_NEIGH):
        rowmin = jnp.min(d2, axis=1, keepdims=True)
        cand = jnp.where(d2 <= rowmin, lane, jnp.int32(n))
        sel = jnp.min(cand, axis=1, keepdims=True)
        sels.append(sel)
        d2 = jnp.where(lane == sel, jnp.float32(1e30), d2)
    idx_ref[0] = jnp.concatenate(sels, axis=1).T + b * n


def _stage1_call(xyzp, features, kernel_W, kernel_b, wq_W, wk_W, wv_W,
                 att1_W, att1_b):
    bsz, n, feat = features.shape
    hid = kernel_W.shape[1]
    xyzt = jnp.transpose(xyzp, (0, 2, 1))
    grid = (bsz, n // M1)
    full = lambda i, j: (i, 0, 0)
    blk = lambda i, j: (i, j, 0)
    idxmap = lambda i, j: (i, 0, j)
    w2d = lambda i, j: (0, 0)
    return pl.pallas_call(
        _stage1_body,
        grid=grid,
        in_specs=[
            pl.BlockSpec((1, M1, 4), blk),
            pl.BlockSpec((1, 4, n), full),
            pl.BlockSpec((1, M1, feat), blk),
            pl.BlockSpec((feat, hid), w2d),
            pl.BlockSpec((1, hid), w2d),
            pl.BlockSpec((hid, hid), w2d),
            pl.BlockSpec((hid, hid), w2d),
            pl.BlockSpec((hid, hid), w2d),
            pl.BlockSpec((hid, hid), w2d),
            pl.BlockSpec((1, hid), w2d),
        ],
        out_specs=[
            pl.BlockSpec((1, M1, TBL), blk),
            pl.BlockSpec((1, M1, hid), blk),
            pl.BlockSpec((1, K_NEIGH, M1), idxmap),
        ],
        out_shape=[
            jax.ShapeDtypeStruct((bsz, n, TBL), jnp.int32),
            jax.ShapeDtypeStruct((bsz, n, hid), jnp.float32),
            jax.ShapeDtypeStruct((bsz, K_NEIGH, n), jnp.int32),
        ],
    )(xyzp, xyzt, features, kernel_W, kernel_b.reshape(1, -1),
      wq_W, wk_W, wv_W, att1_W, att1_b.reshape(1, -1))


def _sc_gather(tmain, idx_flat, n):
    total = idx_flat.shape[0]
    dmain = tmain.shape[1]
    per_w = total // NUM_WORKERS
    nchunks = per_w // GATHER_CHUNK
    mesh = plsc.VectorSubcoreMesh(core_axis_name="c", subcore_axis_name="s")

    @functools.partial(
        pl.kernel,
        out_type=jax.ShapeDtypeStruct((total // n, n, dmain), tmain.dtype),
        mesh=mesh,
        scratch_types=[
            pltpu.VMEM((per_w,), jnp.int32),
            pltpu.VMEM((GATHER_CHUNK, dmain), tmain.dtype),
            pltpu.VMEM((GATHER_CHUNK, dmain), tmain.dtype),
            pltpu.SemaphoreType.DMA,
            pltpu.SemaphoreType.DMA,
            pltpu.SemaphoreType.DMA,
            pltpu.SemaphoreType.DMA,
        ],
    )
    def k(tmain_hbm, idx_hbm, gm_hbm, idx_v, bm0, bm1, sg0, sg1, sw0, sw1):
        wid = lax.axis_index("s") * 2 + lax.axis_index("c")
        base = wid * per_w
        pltpu.sync_copy(idx_hbm.at[pl.ds(base, per_w)], idx_v)
        bm = (bm0, bm1)
        sg = (sg0, sg1)
        sw = (sw0, sw1)
        gh = {}
        wh = {}

        def issue_writeback(c):
            pb = c % 2
            gh[c].wait()
            grow = base + c * GATHER_CHUNK
            wh[c] = pltpu.async_copy(
                bm[pb], gm_hbm.at[grow // n, pl.ds(grow % n, GATHER_CHUNK)],
                sw[pb])

        for c in range(nchunks):
            bb = c % 2
            if c >= 2:
                wh[c - 2].wait()
            isl = idx_v.at[pl.ds(c * GATHER_CHUNK, GATHER_CHUNK)]
            gh[c] = pltpu.async_copy(tmain_hbm.at[isl], bm[bb], sg[bb])
            if c >= 1:
                issue_writeback(c - 1)
        issue_writeback(nchunks - 1)
        wh[nchunks - 2].wait()
        wh[nchunks - 1].wait()

    return k(tmain, idx_flat)


def _stage3_body(g_ref, a_ref, feat_ref, xi_ref,
                 p1W_ref, p1b_ref, p2W_ref, p2b_ref, a1W_ref,
                 a2W_ref, a2b_ref, agW_ref, agb_ref, out_ref):
    hid = a_ref.shape[2]
    a = a_ref[0]
    xi = xi_ref[0]
    atts = []
    wvs = []
    for k in range(K_NEIGH):
        g = g_ref[k]
        w = g[:, :hid]
        kb = _unpack_lo(w)
        vb = _unpack_hi(w)
        xw = g[:, hid:hid + 4]
        xn = _unpack_lo(xw) + _unpack_hi(xw)
        pin = xi - xn
        ph = jnp.maximum(_mm(pin, p1W_ref[...]) + p1b_ref[...], 0.0)
        posenc = _mm(ph, p2W_ref[...]) + p2b_ref[...]
        pre1 = a - kb + _mm(posenc, a1W_ref[...])
        att = _mm(jnp.maximum(pre1, 0.0), a2W_ref[...]) + a2b_ref[...]
        atts.append(att)
        wvs.append(vb + posenc)
    ssum = jnp.zeros_like(atts[0])
    acc = jnp.zeros_like(atts[0])
    for k in range(K_NEIGH):
        e = jnp.exp(atts[k])
        ssum = ssum + e
        acc = acc + e * wvs[k]
    res = acc / ssum
    out_ref[0] = _mm(res, agW_ref[...]) + agb_ref[...] + feat_ref[0]


def _stage3_call(g, a, features, xyzp,
                 pos1_W, pos1_b, pos2_W, pos2_b, att1_W,
                 att2_W, att2_b, agg_W, agg_b):
    bsz, n, feat = features.shape
    hid = pos2_W.shape[0]
    grid = (bsz, n // M3)
    gblk = lambda i, j: (i, j, 0)
    blk = lambda i, j: (i, j, 0)
    w2d = lambda i, j: (0, 0)
    return pl.pallas_call(
        _stage3_body,
        grid=grid,
        in_specs=[
            pl.BlockSpec((K_NEIGH, M3, TBL), gblk),
            pl.BlockSpec((1, M3, hid), blk),
            pl.BlockSpec((1, M3, feat), blk),
            pl.BlockSpec((1, M3, 4), blk),
            pl.BlockSpec((4, hid), w2d),
            pl.BlockSpec((1, hid), w2d),
            pl.BlockSpec((hid, hid), w2d),
            pl.BlockSpec((1, hid), w2d),
            pl.BlockSpec((hid, hid), w2d),
            pl.BlockSpec((hid, hid), w2d),
            pl.BlockSpec((1, hid), w2d),
            pl.BlockSpec((hid, feat), w2d),
            pl.BlockSpec((1, feat), w2d),
        ],
        out_specs=[pl.BlockSpec((1, M3, feat), blk)],
        out_shape=[jax.ShapeDtypeStruct((bsz, n, feat), jnp.float32)],
    )(g, a, features, xyzp,
      pos1_W, pos1_b.reshape(1, -1), pos2_W, pos2_b.reshape(1, -1),
      att1_W, att2_W * (1.0 / 16.0), att2_b.reshape(1, -1) * (1.0 / 16.0),
      agg_W, agg_b.reshape(1, -1))[0]


def kernel(xyzp, features, kernel_W, kernel_b, agg_W, agg_b, wq_W, wk_W,
           wv_W, pos1_W, pos1_b, pos2_W, pos2_b, att1_W, att1_b, att2_W,
           att2_b):
    bsz, n, _ = xyzp.shape

    stage1 = []
    for b in range(bsz):
        xb = lax.slice_in_dim(xyzp, b, b + 1, axis=0)
        fb = lax.slice_in_dim(features, b, b + 1, axis=0)
        tmain, a, idx_t = _stage1_call(
            xb, fb, kernel_W, kernel_b, wq_W, wk_W, wv_W, att1_W, att1_b)
        stage1.append((xb, fb, a, tmain, idx_t))
    gathered = [
        _sc_gather(tmain.reshape(n, TBL), idx_t.reshape(K_NEIGH * n), n)
        for (_, _, _, tmain, idx_t) in stage1]
    outs = [
        _stage3_call(
            g, a, fb, xb,
            pos1_W, pos1_b, pos2_W, pos2_b, att1_W, att2_W, att2_b,
            agg_W, agg_b)
        for g, (xb, fb, a, _, _) in zip(gathered, stage1)]
    return jnp.concatenate(outs, axis=0)


# speedup vs baseline: 1.2762x; 1.2762x over previous
import functools

import jax
import jax.numpy as jnp
from jax import lax
from jax.experimental import pallas as pl
from jax.experimental.pallas import tpu as pltpu
from jax.experimental.pallas import tpu_sc as plsc

K_NEIGH = 16
M1 = 256
M3 = 256
TBL = 384
GATHER_CHUNK = 128
NUM_WORKERS = 32


def _mm(a, b):
    return lax.dot_general(
        a.astype(jnp.bfloat16), b.astype(jnp.bfloat16),
        (((1,), (0,)), ((), ())), preferred_element_type=jnp.float32)


def _pack(lo_bf16, hi_bf16):
    lo = lax.bitcast_convert_type(lo_bf16, jnp.int16).astype(jnp.int32)
    hi = lax.bitcast_convert_type(hi_bf16, jnp.int16).astype(jnp.int32)
    return (lo & jnp.int32(0xFFFF)) | (hi << 16)


def _unpack_lo(w):
    return lax.bitcast_convert_type(
        w.astype(jnp.int16), jnp.bfloat16).astype(jnp.float32)


def _unpack_hi(w):
    return lax.bitcast_convert_type(
        lax.shift_right_logical(w, 16).astype(jnp.int16),
        jnp.bfloat16).astype(jnp.float32)


def _stage1_body(xyzp_ref, xyzt_ref, feat_ref,
                 kW_ref, kb_ref, wq_ref, wk_ref, wv_ref, a1W_ref, a1b_ref,
                 tmain_ref, a_ref, idx_ref):
    n = xyzt_ref.shape[2]
    m = feat_ref.shape[1]
    b = pl.program_id(0)

    fb = feat_ref[0]
    x = _mm(fb, kW_ref[...]) + kb_ref[...]
    q = _mm(x, wq_ref[...])
    a_ref[0] = _mm(q, a1W_ref[...]) + a1b_ref[...]
    kk = _mm(x, wk_ref[...])
    bk = _mm(kk, a1W_ref[...])
    v = _mm(x, wv_ref[...])

    xb = xyzp_ref[0]
    xhi = xb.astype(jnp.bfloat16)
    xlo = (xb - xhi.astype(jnp.float32)).astype(jnp.bfloat16)
    tmain_ref[0] = jnp.concatenate(
        [_pack(bk.astype(jnp.bfloat16), v.astype(jnp.bfloat16)),
         _pack(xhi, xlo),
         jnp.zeros((m, TBL - 260), jnp.int32)], axis=1)

    xyz_i = xb[:, 0:3]
    xyz_jt = xyzt_ref[0][0:3, :]
    sq_i = jnp.sum(xyz_i * xyz_i, axis=1, keepdims=True)
    sq_j = jnp.sum(xyz_jt * xyz_jt, axis=0, keepdims=True)
    d2 = sq_i + sq_j - 2.0 * _mm(xyz_i, xyz_jt)

    ngrp = n // 128
    big = jnp.float32(1e30)
    lane128 = lax.broadcasted_iota(jnp.int32, (m, 128), 1)
    vs = [jnp.full((m, 128), big) for _ in range(4)]
    is_ = [jnp.zeros((m, 128), jnp.int32) for _ in range(4)]
    for j in range(ngrp):
        v = d2[:, j * 128:(j + 1) * 128]
        i = lane128 + j * 128
        ps = [v < vk for vk in vs]
        for t in range(3, 0, -1):
            vs[t] = jnp.where(ps[t], jnp.where(ps[t - 1], vs[t - 1], v),
                              vs[t])
            is_[t] = jnp.where(ps[t], jnp.where(ps[t - 1], is_[t - 1], i),
                               is_[t])
        vs[0] = jnp.where(ps[0], v, vs[0])
        is_[0] = jnp.where(ps[0], i, is_[0])
    cv = jnp.concatenate(vs, axis=1)
    ci = jnp.concatenate(is_, axis=1)
    sels = []
    for _ in range(K_NEIGH):
        rowmin = jnp.min(cv, axis=1, keepdims=True)
        cand = jnp.where(cv <= rowmin, ci, jnp.int32(n))
        sel = jnp.min(cand, axis=1, keepdims=True)
        sels.append(sel)
        cv = jnp.where(ci == sel, big, cv)
    idx_ref[0] = jnp.concatenate(sels, axis=1).T + b * n


def _stage1_call(xyzp, features, kernel_W, kernel_b, wq_W, wk_W, wv_W,
                 att1_W, att1_b):
    bsz, n, feat = features.shape
    hid = kernel_W.shape[1]
    xyzt = jnp.transpose(xyzp, (0, 2, 1))
    grid = (bsz, n // M1)
    full = lambda i, j: (i, 0, 0)
    blk = lambda i, j: (i, j, 0)
    idxmap = lambda i, j: (i, 0, j)
    w2d = lambda i, j: (0, 0)
    return pl.pallas_call(
        _stage1_body,
        grid=grid,
        in_specs=[
            pl.BlockSpec((1, M1, 4), blk),
            pl.BlockSpec((1, 4, n), full),
            pl.BlockSpec((1, M1, feat), blk),
            pl.BlockSpec((feat, hid), w2d),
            pl.BlockSpec((1, hid), w2d),
            pl.BlockSpec((hid, hid), w2d),
            pl.BlockSpec((hid, hid), w2d),
            pl.BlockSpec((hid, hid), w2d),
            pl.BlockSpec((hid, hid), w2d),
            pl.BlockSpec((1, hid), w2d),
        ],
        out_specs=[
            pl.BlockSpec((1, M1, TBL), blk),
            pl.BlockSpec((1, M1, hid), blk),
            pl.BlockSpec((1, K_NEIGH, M1), idxmap),
        ],
        out_shape=[
            jax.ShapeDtypeStruct((bsz, n, TBL), jnp.int32),
            jax.ShapeDtypeStruct((bsz, n, hid), jnp.float32),
            jax.ShapeDtypeStruct((bsz, K_NEIGH, n), jnp.int32),
        ],
    )(xyzp, xyzt, features, kernel_W, kernel_b.reshape(1, -1),
      wq_W, wk_W, wv_W, att1_W, att1_b.reshape(1, -1))


def _sc_gather(tmain, idx_flat, n):
    total = idx_flat.shape[0]
    dmain = tmain.shape[1]
    per_w = total // NUM_WORKERS
    nchunks = per_w // GATHER_CHUNK
    mesh = plsc.VectorSubcoreMesh(core_axis_name="c", subcore_axis_name="s")

    @functools.partial(
        pl.kernel,
        out_type=jax.ShapeDtypeStruct((total // n, n, dmain), tmain.dtype),
        mesh=mesh,
        scratch_types=[
            pltpu.VMEM((per_w,), jnp.int32),
            pltpu.VMEM((GATHER_CHUNK, dmain), tmain.dtype),
            pltpu.VMEM((GATHER_CHUNK, dmain), tmain.dtype),
            pltpu.SemaphoreType.DMA,
            pltpu.SemaphoreType.DMA,
            pltpu.SemaphoreType.DMA,
            pltpu.SemaphoreType.DMA,
        ],
    )
    def k(tmain_hbm, idx_hbm, gm_hbm, idx_v, bm0, bm1, sg0, sg1, sw0, sw1):
        wid = lax.axis_index("s") * 2 + lax.axis_index("c")
        base = wid * per_w
        pltpu.sync_copy(idx_hbm.at[pl.ds(base, per_w)], idx_v)
        bm = (bm0, bm1)
        sg = (sg0, sg1)
        sw = (sw0, sw1)
        gh = {}
        wh = {}

        def issue_writeback(c):
            pb = c % 2
            gh[c].wait()
            grow = base + c * GATHER_CHUNK
            wh[c] = pltpu.async_copy(
                bm[pb], gm_hbm.at[grow // n, pl.ds(grow % n, GATHER_CHUNK)],
                sw[pb])

        for c in range(nchunks):
            bb = c % 2
            if c >= 2:
                wh[c - 2].wait()
            isl = idx_v.at[pl.ds(c * GATHER_CHUNK, GATHER_CHUNK)]
            gh[c] = pltpu.async_copy(tmain_hbm.at[isl], bm[bb], sg[bb])
            if c >= 1:
                issue_writeback(c - 1)
        issue_writeback(nchunks - 1)
        wh[nchunks - 2].wait()
        wh[nchunks - 1].wait()

    return k(tmain, idx_flat)


def _stage3_body(g_ref, a_ref, feat_ref, xi_ref,
                 p1W_ref, p1b_ref, p2W_ref, p2b_ref, a1W_ref,
                 a2W_ref, a2b_ref, agW_ref, agb_ref, out_ref):
    hid = a_ref.shape[2]
    a = a_ref[0]
    xi = xi_ref[0]
    atts = []
    wvs = []
    for k in range(K_NEIGH):
        g = g_ref[k]
        w = g[:, :hid]
        kb = _unpack_lo(w)
        vb = _unpack_hi(w)
        xw = g[:, hid:hid + 4]
        xn = _unpack_lo(xw) + _unpack_hi(xw)
        pin = xi - xn
        ph = jnp.maximum(_mm(pin, p1W_ref[...]) + p1b_ref[...], 0.0)
        posenc = _mm(ph, p2W_ref[...]) + p2b_ref[...]
        pre1 = a - kb + _mm(posenc, a1W_ref[...])
        att = _mm(jnp.maximum(pre1, 0.0), a2W_ref[...]) + a2b_ref[...]
        atts.append(att)
        wvs.append(vb + posenc)
    ssum = jnp.zeros_like(atts[0])
    acc = jnp.zeros_like(atts[0])
    for k in range(K_NEIGH):
        e = jnp.exp(atts[k])
        ssum = ssum + e
        acc = acc + e * wvs[k]
    res = acc / ssum
    out_ref[0] = _mm(res, agW_ref[...]) + agb_ref[...] + feat_ref[0]


def _stage3_call(g, a, features, xyzp,
                 pos1_W, pos1_b, pos2_W, pos2_b, att1_W,
                 att2_W, att2_b, agg_W, agg_b):
    bsz, n, feat = features.shape
    hid = pos2_W.shape[0]
    grid = (bsz, n // M3)
    gblk = lambda i, j: (i, j, 0)
    blk = lambda i, j: (i, j, 0)
    w2d = lambda i, j: (0, 0)
    return pl.pallas_call(
        _stage3_body,
        grid=grid,
        in_specs=[
            pl.BlockSpec((K_NEIGH, M3, TBL), gblk),
            pl.BlockSpec((1, M3, hid), blk),
            pl.BlockSpec((1, M3, feat), blk),
            pl.BlockSpec((1, M3, 4), blk),
            pl.BlockSpec((4, hid), w2d),
            pl.BlockSpec((1, hid), w2d),
            pl.BlockSpec((hid, hid), w2d),
            pl.BlockSpec((1, hid), w2d),
            pl.BlockSpec((hid, hid), w2d),
            pl.BlockSpec((hid, hid), w2d),
            pl.BlockSpec((1, hid), w2d),
            pl.BlockSpec((hid, feat), w2d),
            pl.BlockSpec((1, feat), w2d),
        ],
        out_specs=[pl.BlockSpec((1, M3, feat), blk)],
        out_shape=[jax.ShapeDtypeStruct((bsz, n, feat), jnp.float32)],
    )(g, a, features, xyzp,
      pos1_W, pos1_b.reshape(1, -1), pos2_W, pos2_b.reshape(1, -1),
      att1_W, att2_W * (1.0 / 16.0), att2_b.reshape(1, -1) * (1.0 / 16.0),
      agg_W, agg_b.reshape(1, -1))[0]


def kernel(xyzp, features, kernel_W, kernel_b, agg_W, agg_b, wq_W, wk_W,
           wv_W, pos1_W, pos1_b, pos2_W, pos2_b, att1_W, att1_b, att2_W,
           att2_b):
    bsz, n, _ = xyzp.shape

    stage1 = []
    for b in range(bsz):
        xb = lax.slice_in_dim(xyzp, b, b + 1, axis=0)
        fb = lax.slice_in_dim(features, b, b + 1, axis=0)
        tmain, a, idx_t = _stage1_call(
            xb, fb, kernel_W, kernel_b, wq_W, wk_W, wv_W, att1_W, att1_b)
        stage1.append((xb, fb, a, tmain, idx_t))
    gathered = [
        _sc_gather(tmain.reshape(n, TBL), idx_t.reshape(K_NEIGH * n), n)
        for (_, _, _, tmain, idx_t) in stage1]
    outs = [
        _stage3_call(
            g, a, fb, xb,
            pos1_W, pos1_b, pos2_W, pos2_b, att1_W, att2_W, att2_b,
            agg_W, agg_b)
        for g, (xb, fb, a, _, _) in zip(gathered, stage1)]
    return jnp.concatenate(outs, axis=0)
